# Initial kernel scaffold; baseline (speedup 1.0000x reference)
import functools

import jax
import jax.numpy as jnp
from jax import lax
from jax.experimental import pallas as pl
from jax.experimental.pallas import tpu as pltpu
from jax.experimental.pallas import tpu_sc as plsc

NC = 2
NT = 16
LANES = 16
CHUNK = 128
IDX_BLK = 24
DH = 64


def _build_sc_call(n, n_pad, n_chunks):
  acc_rows = 2 * n_pad
  cpt = n_chunks // NT
  nblk = cpt // IDX_BLK
  rows_per_tile = acc_rows // NT

  mesh = plsc.VectorSubcoreMesh(core_axis_name="c", subcore_axis_name="s")

  @functools.partial(
      pl.kernel,
      mesh=mesh,
      out_type=jax.ShapeDtypeStruct((NC * acc_rows, DH), jnp.float32),
      scratch_types=[
          pltpu.VMEM((IDX_BLK, 2, CHUNK), jnp.int32),
          pltpu.VMEM((CHUNK,), jnp.int32),
          pltpu.VMEM((CHUNK,), jnp.int32),
          pltpu.VMEM((CHUNK, DH), jnp.float32),
          pltpu.VMEM((CHUNK, DH), jnp.float32),
          pltpu.VMEM_SHARED((acc_rows, DH), jnp.float32),
          pltpu.SemaphoreType.DMA,
          pltpu.SemaphoreType.DMA,
      ],
  )
  def sc_kernel(x_hbm, arr_hbm, out_hbm,
                idxblk, srcb0, srcb1, rows0, rows1, acc, sem0, sem1):
    cid = lax.axis_index("c")
    tid = lax.axis_index("s")
    srcbs = (srcb0, srcb1)
    rowsb = (rows0, rows1)
    sems = (sem0, sem1)

    def zrow(r, carry):
      for j in range(DH // LANES):
        rows0[r, pl.ds(j * LANES, LANES)] = jnp.zeros((LANES,), jnp.float32)
      return carry

    lax.fori_loop(0, CHUNK, zrow, 0)

    def zcp(q, carry):
      pltpu.sync_copy(rows0, acc.at[pl.ds(tid * rows_per_tile + q * CHUNK, CHUNK)])
      return carry

    lax.fori_loop(0, rows_per_tile // CHUNK, zcp, 0)
    plsc.subcore_barrier()

    row_base = cid * n

    def build_src(k, b):
      for j in range(CHUNK // LANES):
        v = idxblk[k, 0, pl.ds(j * LANES, LANES)]
        srcbs[b][pl.ds(j * LANES, LANES)] = v + row_base

    def start_gather(b):
      pltpu.make_async_copy(x_hbm.at[srcbs[b]], rowsb[b], sems[b]).start()

    def wait_gather(b):
      pltpu.make_async_copy(x_hbm.at[srcbs[b]], rowsb[b], sems[b]).wait()

    def scatter_add(k, b):
      pltpu.sync_copy(rowsb[b], acc.at[idxblk.at[k, 1]], add=True)

    c0 = tid * cpt

    def blk_body(bi, carry):
      blk_start = c0 + bi * IDX_BLK
      pltpu.sync_copy(arr_hbm.at[pl.ds(blk_start, IDX_BLK)], idxblk)
      build_src(0, 0)
      start_gather(0)

      def pair_body(p, c2):
        k0 = 2 * p
        build_src(k0 + 1, 1)
        start_gather(1)
        wait_gather(0)
        scatter_add(k0, 0)

        @pl.when(k0 + 2 < IDX_BLK)
        def _():
          build_src(k0 + 2, 0)
          start_gather(0)

        wait_gather(1)
        scatter_add(k0 + 1, 1)
        return c2

      lax.fori_loop(0, IDX_BLK // 2, pair_body, 0)
      return carry

    lax.fori_loop(0, nblk, blk_body, 0)

    plsc.subcore_barrier()
    out_row0 = cid * acc_rows + tid * rows_per_tile
    pltpu.sync_copy(acc.at[pl.ds(tid * rows_per_tile, rows_per_tile)],
                    out_hbm.at[pl.ds(out_row0, rows_per_tile)])

  return sc_kernel, acc_rows


def kernel(x, edge_index, edge_index2):
  n, d = x.shape
  assert d == 2 * DH
  n_pad = ((n + 1023) // 1024) * 1024
  dummy = n_pad - 1

  x2h = jnp.concatenate([x[:, :DH], x[:, DH:]], axis=0)

  src = jnp.concatenate([edge_index[1], edge_index2[1]])
  dst = jnp.concatenate([edge_index[0], edge_index2[0] + n_pad])
  e_tot = src.shape[0]
  grain = NT * IDX_BLK * CHUNK
  e_pad = ((e_tot + grain - 1) // grain) * grain
  src = jnp.pad(src, (0, e_pad - e_tot))
  dst = jnp.pad(dst, (0, e_pad - e_tot), constant_values=dummy)
  n_chunks = e_pad // CHUNK
  arr = jnp.stack([src.reshape(n_chunks, CHUNK), dst.reshape(n_chunks, CHUNK)],
                  axis=1)

  sc_call, acc_rows = _build_sc_call(n, n_pad, n_chunks)
  o = sc_call(x2h, arr)
  o0, o1 = o[:acc_rows], o[acc_rows:]
  x1 = jnp.concatenate([o0[:n], o1[:n]], axis=1)
  x2 = jnp.concatenate([o0[n_pad:n_pad + n], o1[n_pad:n_pad + n]], axis=1)
  return jnp.concatenate([x1, x2], axis=1)


# speedup vs baseline: 10.1902x; 10.1902x over previous
import functools

import jax
import jax.numpy as jnp
from jax import lax
from jax.experimental import pallas as pl
from jax.experimental.pallas import tpu as pltpu
from jax.experimental.pallas import tpu_sc as plsc

NC = 2
NT = 16
LANES = 16
CHUNK = 128
IDX_BLK = 24
DH = 64


def _build_sc_call(n, n_pad, n_chunks):
  acc_rows = 2 * n_pad
  cpt = n_chunks // NT
  nblk = cpt // IDX_BLK
  rows_per_tile = acc_rows // NT

  mesh = plsc.VectorSubcoreMesh(core_axis_name="c", subcore_axis_name="s")

  @functools.partial(
      pl.kernel,
      mesh=mesh,
      compiler_params=pltpu.CompilerParams(use_tc_tiling_on_sc=False),
      out_type=jax.ShapeDtypeStruct((NC * acc_rows, DH), jnp.float32),
      scratch_types=[
          pltpu.VMEM((IDX_BLK, 2, CHUNK), jnp.int32),
          pltpu.VMEM((CHUNK,), jnp.int32),
          pltpu.VMEM((CHUNK,), jnp.int32),
          pltpu.VMEM((CHUNK, DH), jnp.float32),
          pltpu.VMEM((CHUNK, DH), jnp.float32),
          pltpu.VMEM_SHARED((acc_rows, DH), jnp.float32),
          pltpu.SemaphoreType.DMA,
          pltpu.SemaphoreType.DMA,
      ],
  )
  def sc_kernel(x_hbm, arr_hbm, out_hbm,
                idxblk, srcb0, srcb1, rows0, rows1, acc, sem0, sem1):
    cid = lax.axis_index("c")
    tid = lax.axis_index("s")
    srcbs = (srcb0, srcb1)
    rowsb = (rows0, rows1)
    sems = (sem0, sem1)

    def zrow(r, carry):
      for j in range(DH // LANES):
        rows0[r, pl.ds(j * LANES, LANES)] = jnp.zeros((LANES,), jnp.float32)
      return carry

    lax.fori_loop(0, CHUNK, zrow, 0)

    def zcp(q, carry):
      pltpu.sync_copy(rows0, acc.at[pl.ds(tid * rows_per_tile + q * CHUNK, CHUNK)])
      return carry

    lax.fori_loop(0, rows_per_tile // CHUNK, zcp, 0)
    plsc.subcore_barrier()

    row_base = cid * n

    def build_src(k, b):
      for j in range(CHUNK // LANES):
        v = idxblk[k, 0, pl.ds(j * LANES, LANES)]
        srcbs[b][pl.ds(j * LANES, LANES)] = v + row_base

    def start_gather(b):
      pltpu.make_async_copy(x_hbm.at[srcbs[b]], rowsb[b], sems[b]).start()

    def wait_gather(b):
      pltpu.make_async_copy(x_hbm.at[srcbs[b]], rowsb[b], sems[b]).wait()

    def scatter_add(k, b):
      pltpu.sync_copy(rowsb[b], acc.at[idxblk.at[k, 1]], add=True)

    c0 = tid * cpt

    def blk_body(bi, carry):
      blk_start = c0 + bi * IDX_BLK
      pltpu.sync_copy(arr_hbm.at[pl.ds(blk_start, IDX_BLK)], idxblk)
      build_src(0, 0)
      start_gather(0)

      def pair_body(p, c2):
        k0 = 2 * p
        build_src(k0 + 1, 1)
        start_gather(1)
        wait_gather(0)
        scatter_add(k0, 0)

        @pl.when(k0 + 2 < IDX_BLK)
        def _():
          build_src(k0 + 2, 0)
          start_gather(0)

        wait_gather(1)
        scatter_add(k0 + 1, 1)
        return c2

      lax.fori_loop(0, IDX_BLK // 2, pair_body, 0)
      return carry

    lax.fori_loop(0, nblk, blk_body, 0)

    plsc.subcore_barrier()
    out_row0 = cid * acc_rows + tid * rows_per_tile
    pltpu.sync_copy(acc.at[pl.ds(tid * rows_per_tile, rows_per_tile)],
                    out_hbm.at[pl.ds(out_row0, rows_per_tile)])

  return sc_kernel, acc_rows


def kernel(x, edge_index, edge_index2):
  n, d = x.shape
  assert d == 2 * DH
  n_pad = ((n + 1023) // 1024) * 1024
  dummy = n_pad - 1

  x2h = jnp.concatenate([x[:, :DH], x[:, DH:]], axis=0)

  src = jnp.concatenate([edge_index[1], edge_index2[1]])
  dst = jnp.concatenate([edge_index[0], edge_index2[0] + n_pad])
  e_tot = src.shape[0]
  grain = NT * IDX_BLK * CHUNK
  e_pad = ((e_tot + grain - 1) // grain) * grain
  src = jnp.pad(src, (0, e_pad - e_tot))
  dst = jnp.pad(dst, (0, e_pad - e_tot), constant_values=dummy)
  n_chunks = e_pad // CHUNK
  arr = jnp.stack([src.reshape(n_chunks, CHUNK), dst.reshape(n_chunks, CHUNK)],
                  axis=1)

  sc_call, acc_rows = _build_sc_call(n, n_pad, n_chunks)
  o = sc_call(x2h, arr)
  o0, o1 = o[:acc_rows], o[acc_rows:]
  x1 = jnp.concatenate([o0[:n], o1[:n]], axis=1)
  x2 = jnp.concatenate([o0[n_pad:n_pad + n], o1[n_pad:n_pad + n]], axis=1)
  return jnp.concatenate([x1, x2], axis=1)
